# Initial kernel scaffold; baseline (speedup 1.0000x reference)
import jax
import jax.numpy as jnp
from jax.experimental import pallas as pl


def kernel(x, f, kernelA, W1p, b1p, W2p, b2p):
    raise NotImplementedError("write your pallas kernel here")


# speedup vs baseline: 7.2514x; 7.2514x over previous
import functools

import jax
import jax.numpy as jnp
from jax import lax
from jax.experimental import pallas as pl
from jax.experimental.pallas import tpu as pltpu

_ML = 3
_K = 7
_P = _K // 2


def _rup(v, m):
    return ((v + m - 1) // m) * m


def _taps_mlp_kernel(x_ref, w1_ref, b1_ref, w2_ref, b2_ref, o_ref):
    h = jnp.dot(x_ref[...], w1_ref[...], preferred_element_type=jnp.float32)
    h = jnp.maximum(h + b1_ref[...], 0.0)
    o_ref[...] = (
        jnp.dot(h, w2_ref[...], preferred_element_type=jnp.float32) + b2_ref[...]
    )


def _shift_rows(arr, k, S, Bc):
    if k == 0:
        return arr
    z = jnp.zeros((abs(k), Bc), jnp.float32)
    if k > 0:
        return jnp.concatenate([arr[k:, :], z], axis=0)
    return jnp.concatenate([z, arr[:k, :]], axis=0)


def _conv_pass(read, write, taps_ref, tap_base, K, N, S, Bc):
    def body(i, _):
        base = pl.multiple_of((i + _P) * S, S)
        srcs = [
            read(pl.multiple_of(base + (ay - _P) * S, S)) for ay in range(K)
        ]
        acc = None
        for j in range(K):
            pj = None
            for ay in range(K):
                w = taps_ref[tap_base + ay * K + j, :][None, :]
                term = w * srcs[ay]
                pj = term if pj is None else pj + term
            pj = _shift_rows(pj, j - _P, S, Bc)
            acc = pj if acc is None else acc + pj
        write(base, acc)
        return ()

    lax.fori_loop(0, N, body, (), unroll=False)


def _smoother_kernel(taps_ref, x_hbm, f_hbm, o_hbm, xs, fs, tmp, outs,
                     sem_x, sem_f, sem_o, *, N, S, Bc):
    c = pl.program_id(0)
    lane0 = pl.multiple_of(c * Bc, Bc)
    cp_x = pltpu.make_async_copy(x_hbm.at[:, pl.ds(lane0, Bc)], xs, sem_x)
    cp_f = pltpu.make_async_copy(f_hbm.at[:, pl.ds(lane0, Bc)], fs, sem_f)
    cp_x.start()
    cp_f.start()

    rows = (N + 2 * _P) * S
    tmp[0 : _P * S, :] = jnp.zeros((_P * S, Bc), jnp.float32)
    tmp[(N + _P) * S : rows, :] = jnp.zeros((_P * S, Bc), jnp.float32)

    ri = lax.broadcasted_iota(jnp.int32, (S, Bc), 0)
    colmask = (ri >= _P) & (ri < N + _P)

    cp_x.wait()
    cp_f.wait()

    def body_a(i, _):
        base = pl.multiple_of((i + _P) * S, S)
        srcs = [
            xs[pl.ds(pl.multiple_of(base + (ay - 1) * S, S), S), :]
            for ay in range(3)
        ]
        acc = None
        for j in range(3):
            pj = None
            for ay in range(3):
                w = taps_ref[ay * 3 + j, :][None, :]
                term = w * srcs[ay]
                pj = term if pj is None else pj + term
            pj = _shift_rows(pj, j - 1, S, Bc)
            acc = pj if acc is None else acc + pj
        fband = fs[pl.ds(base, S), :]
        fs[pl.ds(base, S), :] = jnp.where(colmask, fband - acc, 0.0)
        return ()

    lax.fori_loop(0, N, body_a, (), unroll=False)

    outs[...] = xs[...]

    for ch in range(_ML):
        base1 = 9 + ch * _K * _K
        base2 = 9 + _ML * _K * _K + ch * _K * _K

        def c1_read(row):
            return fs[pl.ds(row, S), :]

        def c1_write(base, val):
            tmp[pl.ds(base, S), :] = jnp.where(colmask, val, 0.0)

        _conv_pass(c1_read, c1_write, taps_ref, base1, _K, N, S, Bc)

        def c2_read(row):
            return tmp[pl.ds(row, S), :]

        def c2_write(base, val):
            oband = outs[pl.ds(base, S), :]
            outs[pl.ds(base, S), :] = oband + val

        _conv_pass(c2_read, c2_write, taps_ref, base2, _K, N, S, Bc)

    cp_o = pltpu.make_async_copy(outs, o_hbm.at[:, pl.ds(lane0, Bc)], sem_o)
    cp_o.start()
    cp_o.wait()


def kernel(x, f, kernelA, W1p, b1p, W2p, b2p):
    B, _, N, _ = x.shape
    dout = _ML * _K * _K
    D = 9 + 2 * dout

    kA_flat = kernelA.reshape(B, 9).astype(jnp.float32)
    dinp = W1p.shape[0]
    doutp = W2p.shape[1]
    Bp = _rup(max(B, 8), 8)
    xp = jnp.zeros((Bp, dinp), jnp.float32).at[:B, :9].set(kA_flat)
    mlp_out = pl.pallas_call(
        _taps_mlp_kernel,
        out_shape=jax.ShapeDtypeStruct((Bp, doutp), jnp.float32),
    )(xp, W1p, b1p, W2p, b2p)
    taps_all = jnp.concatenate([kA_flat, mlp_out[:B, : 2 * dout]], axis=1)

    Bc = 128
    nch = -(-B // Bc)
    Bpad = nch * Bc
    if N + 2 * _P > 128:
        raise ValueError("padded plane wider than 128 sublane-stride unsupported")
    S = 128
    rows = (N + 2 * _P) * S

    Dp = _rup(D, 8)
    tapsT = (
        jnp.zeros((Bpad, Dp), jnp.float32).at[:B, :D].set(taps_all).T
    )

    def to_pos_layout(a):
        ap = jnp.pad(
            a[:, 0].astype(jnp.float32),
            ((0, Bpad - B), (_P, _P), (_P, S - N - _P)),
        )
        return ap.reshape(Bpad, rows).T

    xT = to_pos_layout(x)
    fT = to_pos_layout(f)

    kfn = functools.partial(_smoother_kernel, N=N, S=S, Bc=Bc)
    outT = pl.pallas_call(
        kfn,
        out_shape=jax.ShapeDtypeStruct((rows, Bpad), jnp.float32),
        grid=(nch,),
        in_specs=[
            pl.BlockSpec((Dp, Bc), lambda i: (0, i)),
            pl.BlockSpec(memory_space=pl.ANY),
            pl.BlockSpec(memory_space=pl.ANY),
        ],
        out_specs=pl.BlockSpec(memory_space=pl.ANY),
        scratch_shapes=[
            pltpu.VMEM((rows, Bc), jnp.float32),
            pltpu.VMEM((rows, Bc), jnp.float32),
            pltpu.VMEM((rows, Bc), jnp.float32),
            pltpu.VMEM((rows, Bc), jnp.float32),
            pltpu.SemaphoreType.DMA,
            pltpu.SemaphoreType.DMA,
            pltpu.SemaphoreType.DMA,
        ],
        compiler_params=pltpu.CompilerParams(
            dimension_semantics=("parallel",),
            vmem_limit_bytes=48 * 1024 * 1024,
        ),
    )(tapsT, xT, fT)

    out = outT.T.reshape(Bpad, N + 2 * _P, S)[:B, _P : N + _P, _P : N + _P]
    return out[:, None, :, :]
